# Initial kernel scaffold; baseline (speedup 1.0000x reference)
import jax
import jax.numpy as jnp
from jax.experimental import pallas as pl


def kernel(x, edge_attr, edge_index, Wl1, bl1, Wr1, br1, We1, att1, bias1, g1, be1, Wl2, bl2, Wr2, br2, We2, att2, bias2, g2, be2):
    raise NotImplementedError("write your pallas kernel here")


# speedup vs baseline: 7.7569x; 7.7569x over previous
import functools

import jax
import jax.numpy as jnp
from jax.experimental import pallas as pl

N = 50000
E = 800000
HID = 64
HEADS = 4
HC = 16

_BLK = 2000
_GRID = N // _BLK


def _node_proj_body(x_ref, wl_ref, bl_ref, wr_ref, br_ref, xl_ref, xr_ref):
    x = x_ref[...]
    xl_ref[...] = jnp.dot(x, wl_ref[...], preferred_element_type=jnp.float32) + bl_ref[...]
    xr_ref[...] = jnp.dot(x, wr_ref[...], preferred_element_type=jnp.float32) + br_ref[...]


def _node_proj(x, Wl, bl, Wr, br):
    d_in = x.shape[1]
    d_out = Wl.shape[1]
    full = lambda *s: pl.BlockSpec(s, lambda i: (0,) * len(s))
    return pl.pallas_call(
        _node_proj_body,
        grid=(_GRID,),
        in_specs=[
            pl.BlockSpec((_BLK, d_in), lambda i: (i, 0)),
            full(d_in, d_out), full(1, d_out), full(d_in, d_out), full(1, d_out),
        ],
        out_specs=[pl.BlockSpec((_BLK, d_out), lambda i: (i, 0))] * 2,
        out_shape=[jax.ShapeDtypeStruct((N, d_out), jnp.float32)] * 2,
    )(x, Wl, bl.reshape(1, -1), Wr, br.reshape(1, -1))


def _node_pass1_body(num_ref, den_ref, cntea_ref, xl_ref, xr_ref,
                     we_ref, att_ref, bias_ref, g_ref, be_ref,
                     wl2_ref, bl2_ref, wr2_ref, br2_ref,
                     xl2_ref, xr2_ref):
    cntea = cntea_ref[...]
    cnt = jnp.maximum(cntea[:, 0:1], 1.0)
    lattr = cntea[:, 1:4] / cnt
    eat = jnp.dot(lattr, we_ref[...], preferred_element_type=jnp.float32)
    xl = xl_ref[...]
    m = xl + xr_ref[...] + eat
    m = jnp.where(m > 0, m, 0.2 * m)
    t = m * att_ref[...]
    num = num_ref[...]
    den = den_ref[...]
    outs = []
    for h in range(HEADS):
        sl = slice(h * HC, (h + 1) * HC)
        alpha = jnp.sum(t[:, sl], axis=1, keepdims=True)
        ex = jnp.exp(alpha)
        numf = num[:, sl] + ex * xl[:, sl]
        denf = den[:, h:h + 1] + ex
        outs.append(numf / (denf + 1e-16))
    o = jnp.concatenate(outs, axis=1) + bias_ref[...]
    mu = jnp.mean(o, axis=1, keepdims=True)
    var = jnp.mean((o - mu) ** 2, axis=1, keepdims=True)
    o = (o - mu) * jax.lax.rsqrt(var + 1e-5) * g_ref[...] + be_ref[...]
    o = jnp.where(o > 0, o, jnp.exp(jnp.minimum(o, 0.0)) - 1.0)
    xl2_ref[...] = jnp.dot(o, wl2_ref[...], preferred_element_type=jnp.float32) + bl2_ref[...]
    xr2_ref[...] = jnp.dot(o, wr2_ref[...], preferred_element_type=jnp.float32) + br2_ref[...]


def _node_pass1(num, den, cntea, xl1, xr1, We1, att1, bias1, g1, be1,
                Wl2, bl2, Wr2, br2):
    full = lambda *s: pl.BlockSpec(s, lambda i: (0,) * len(s))
    blk64 = pl.BlockSpec((_BLK, 64), lambda i: (i, 0))
    blk4 = pl.BlockSpec((_BLK, 4), lambda i: (i, 0))
    return pl.pallas_call(
        _node_pass1_body,
        grid=(_GRID,),
        in_specs=[
            blk64, blk4, blk4, blk64, blk64,
            full(3, 64), full(1, 64), full(1, 64), full(1, 64), full(1, 64),
            full(64, 64), full(1, 64), full(64, 64), full(1, 64),
        ],
        out_specs=[blk64, blk64],
        out_shape=[jax.ShapeDtypeStruct((N, 64), jnp.float32)] * 2,
    )(num, den, cntea, xl1, xr1,
      We1, att1.reshape(1, 64), bias1.reshape(1, 64), g1.reshape(1, 64),
      be1.reshape(1, 64), Wl2, bl2.reshape(1, 64), Wr2, br2.reshape(1, 64))


def _node_pass2_body(num_ref, den_ref, cntea_ref, xl_ref, xr_ref,
                     we_ref, att_ref, bias_ref, g_ref, be_ref, out_ref):
    cntea = cntea_ref[...]
    cnt = jnp.maximum(cntea[:, 0:1], 1.0)
    lattr = cntea[:, 1:4] / cnt
    eat = jnp.dot(lattr, we_ref[...], preferred_element_type=jnp.float32)
    xl = xl_ref[...]
    m = xl + xr_ref[...] + eat
    m = jnp.where(m > 0, m, 0.2 * m)
    alpha = jnp.sum(m * att_ref[...], axis=1, keepdims=True)
    ex = jnp.exp(alpha)
    numf = num_ref[...] + ex * xl
    denf = den_ref[...][:, 0:1] + ex
    o = numf / (denf + 1e-16) + bias_ref[...]
    mu = jnp.mean(o, axis=1, keepdims=True)
    var = jnp.mean((o - mu) ** 2, axis=1, keepdims=True)
    out_ref[...] = (o - mu) * jax.lax.rsqrt(var + 1e-5) * g_ref[...] + be_ref[...]


def _node_pass2(num, den, cntea, xl2, xr2, We2, att2, bias2, g2, be2):
    full = lambda *s: pl.BlockSpec(s, lambda i: (0,) * len(s))
    blk64 = pl.BlockSpec((_BLK, 64), lambda i: (i, 0))
    blk4 = pl.BlockSpec((_BLK, 4), lambda i: (i, 0))
    return pl.pallas_call(
        _node_pass2_body,
        grid=(_GRID,),
        in_specs=[
            blk64, blk4, blk4, blk64, blk64,
            full(3, 64), full(1, 64), full(1, 64), full(1, 64), full(1, 64),
        ],
        out_specs=blk64,
        out_shape=jax.ShapeDtypeStruct((N, 64), jnp.float32),
    )(num, den, cntea, xl2, xr2,
      We2, att2.reshape(1, 64), bias2.reshape(1, 64), g2.reshape(1, 64),
      be2.reshape(1, 64))


def _edge_pass_jnp(xl, xr, src, dst, ea, We, att_flat, heads):
    eat = ea @ We
    m = xl[src] + xr[dst] + eat
    m = jnp.where(m > 0, m, 0.2 * m)
    t = m * att_flat[None, :]
    hc = 64 // heads
    alpha = t.reshape(E, heads, hc).sum(-1)
    ex = jnp.exp(alpha)
    num = jax.ops.segment_sum(
        (ex[:, :, None] * xl[src].reshape(E, heads, hc)).reshape(E, 64),
        dst, num_segments=N)
    den = jax.ops.segment_sum(ex, dst, num_segments=N)
    return num, den


def kernel(x, edge_attr, edge_index, Wl1, bl1, Wr1, br1, We1, att1, bias1,
           g1, be1, Wl2, bl2, Wr2, br2, We2, att2, bias2, g2, be2):
    src, dst = edge_index[0], edge_index[1]
    xl1, xr1 = _node_proj(x, Wl1, bl1, Wr1, br1)

    cnt = jax.ops.segment_sum(jnp.ones((E,), jnp.float32), dst, num_segments=N)
    easum = jax.ops.segment_sum(edge_attr, dst, num_segments=N)
    cntea = jnp.concatenate([cnt[:, None], easum], axis=1)

    num1, den1 = _edge_pass_jnp(xl1, xr1, src, dst, edge_attr, We1,
                                att1.reshape(64), HEADS)
    xl2, xr2 = _node_pass1(num1, den1, cntea, xl1, xr1, We1, att1, bias1,
                           g1, be1, Wl2, bl2, Wr2, br2)
    num2, den2 = _edge_pass_jnp(xl2, xr2, src, dst, edge_attr, We2,
                                att2.reshape(64), 1)
    den2 = jnp.concatenate([den2, jnp.zeros((N, 3), jnp.float32)], axis=1)
    return _node_pass2(num2, den2, cntea, xl2, xr2, We2, att2, bias2, g2, be2)


# speedup vs baseline: 11.0734x; 1.4276x over previous
import functools

import numpy as np

import jax
import jax.numpy as jnp
from jax import lax
from jax.experimental import pallas as pl
from jax.experimental.pallas import tpu as pltpu
from jax.experimental.pallas import tpu_sc as plsc

N = 50000
E = 800000
HEADS = 4
HC = 16

_BLK = 2000
_GRID = N // _BLK

K = 96
NTILES = 16
CHUNKS = 522
EPT = K * CHUNKS
E_PAD = EPT * NTILES
ROWS_PT = 3128
N_PAD = ROWS_PT * NTILES


def _make_edge_pass(heads):
    mesh = plsc.VectorSubcoreMesh(core_axis_name="c", subcore_axis_name="s",
                                  num_cores=2, num_subcores=16)

    def body(src_hbm, dst_hbm, eat_hbm, xl_hbm, xr_hbm, const_hbm,
             zero_hbm,
             onum0, onum1, exd_hbm,
             src_v, dst_v, consts_v, eat_v, xlg, xrg, stage, stage_exd, acc,
             sem):
        cid = lax.axis_index("c")
        sid = lax.axis_index("s")
        row0 = pl.multiple_of(sid * ROWS_PT, 64)

        pltpu.sync_copy(zero_hbm.at[pl.ds(row0, ROWS_PT)],
                        acc.at[pl.ds(row0, ROWS_PT)])
        pltpu.sync_copy(const_hbm, consts_v)

        att = [consts_v[0, pl.ds(h * 16, 16)] for h in range(4)]
        lanes = lax.iota(jnp.int32, 16)
        lf = lax.convert_element_type(lanes, jnp.float32)
        clamp01 = lambda v: jnp.minimum(jnp.maximum(v, 0.0), 1.0)
        oh = [clamp01(1.0 - jnp.abs(lf - float(i))) for i in range(4)]
        c0f = lax.broadcast(
            jnp.float32(1.0) - lax.convert_element_type(cid, jnp.float32),
            (16,))
        c1f = 1.0 - c0f
        ebase = sid * EPT
        shuf = [lanes ^ s for s in (8, 4, 2, 1)]

        def allsum(v):
            for sx in shuf:
                v = v + v.at[sx].get(mode="promise_in_bounds")
            return v

        plsc.subcore_barrier()

        def chunk(ch, carry):
            base = pl.multiple_of(ebase + ch * K, 32)
            pltpu.sync_copy(src_hbm.at[pl.ds(base, K)], src_v)
            pltpu.sync_copy(dst_hbm.at[pl.ds(base, K)], dst_v)
            pltpu.sync_copy(eat_hbm.at[pl.ds(base, K)], eat_v)
            cp1 = pltpu.async_copy(xl_hbm.at[src_v], xlg, sem)
            cp2 = pltpu.async_copy(xr_hbm.at[dst_v], xrg, sem)
            cp1.wait()
            cp2.wait()

            def edge(e, carry2):
                xs = []
                exs = []
                tacc = None
                for h in range(4):
                    xlh = xlg[e, pl.ds(h * 16, 16)]
                    xrh = xrg[e, pl.ds(h * 16, 16)]
                    m = xlh + xrh + eat_v[e, pl.ds(h * 16, 16)]
                    m = jnp.maximum(m, 0.0) + 0.2 * jnp.minimum(m, 0.0)
                    t = m * att[h]
                    xs.append(xlh)
                    if heads == 4:
                        exs.append(jnp.exp(allsum(t)))
                    else:
                        tacc = t if tacc is None else tacc + t
                if heads == 1:
                    ex = jnp.exp(allsum(tacc))
                    exs = [ex, ex, ex, ex]
                na = c0f * (exs[0] * xs[0]) + c1f * (exs[2] * xs[2])
                nb = c0f * (exs[1] * xs[1]) + c1f * (exs[3] * xs[3])
                if heads == 4:
                    exrow = (oh[0] * exs[0] + oh[1] * exs[1]
                             + oh[2] * exs[2] + oh[3] * exs[3])
                else:
                    exrow = oh[0] * exs[0]
                stage[e, pl.ds(0, 16)] = na
                stage[e, pl.ds(16, 16)] = nb
                stage_exd[e, pl.ds(0, 16)] = exrow
                return carry2
            lax.fori_loop(0, K, edge, 0)

            pltpu.sync_copy(stage, acc.at[dst_v], add=True)

            @pl.when(cid == 0)
            def _():
                pltpu.sync_copy(stage_exd, exd_hbm.at[pl.ds(base, K)])
            return carry
        lax.fori_loop(0, CHUNKS, chunk, 0)
        plsc.subcore_barrier()

        rows = pl.ds(row0, ROWS_PT)

        @pl.when(cid == 0)
        def _():
            pltpu.sync_copy(acc.at[rows], onum0.at[rows])

        @pl.when(cid == 1)
        def _():
            pltpu.sync_copy(acc.at[rows], onum1.at[rows])

    f32 = jnp.float32
    return pl.kernel(
        body, mesh=mesh,
        compiler_params=pltpu.CompilerParams(use_tc_tiling_on_sc=False),
        out_type=[jax.ShapeDtypeStruct((N_PAD, 32), f32),
                  jax.ShapeDtypeStruct((N_PAD, 32), f32),
                  jax.ShapeDtypeStruct((E_PAD, 16), f32)],
        scratch_types=[
            pltpu.VMEM((K,), jnp.int32),
            pltpu.VMEM((K,), jnp.int32),
            pltpu.VMEM((1, 64), f32),
            pltpu.VMEM((K, 64), f32),
            pltpu.VMEM((K, 64), f32),
            pltpu.VMEM((K, 64), f32),
            pltpu.VMEM((K, 32), f32),
            pltpu.VMEM((K, 16), f32),
            pltpu.VMEM_SHARED((N_PAD, 32), f32),
            pltpu.SemaphoreType.DMA,
        ],
    )


def _make_extra_pass():
    mesh = plsc.VectorSubcoreMesh(core_axis_name="c", subcore_axis_name="s",
                                  num_cores=2, num_subcores=16)

    def body(dst_hbm, exd_hbm, ea_hbm, zero_hbm,
             oden, ocnt,
             dst_v, exd_v, ea_v, stage, acc, sem):
        cid = lax.axis_index("c")
        sid = lax.axis_index("s")
        row0 = pl.multiple_of(sid * ROWS_PT, 64)

        pltpu.sync_copy(zero_hbm.at[pl.ds(row0, ROWS_PT)],
                        acc.at[pl.ds(row0, ROWS_PT)])

        lanes = lax.iota(jnp.int32, 16)
        lf = lax.convert_element_type(lanes, jnp.float32)
        clamp01 = lambda v: jnp.minimum(jnp.maximum(v, 0.0), 1.0)
        cnt1 = clamp01(1.0 - lf)
        eaoff = jnp.minimum(jnp.maximum(lanes - 1, 0), 3)
        m14f = clamp01(lf) * clamp01(4.0 - lf)
        ebase = sid * EPT

        plsc.subcore_barrier()

        def chunk(ch, carry):
            base = pl.multiple_of(ebase + ch * K, 32)
            base4 = pl.multiple_of((ebase + ch * K) // 4, 8)
            pltpu.sync_copy(dst_hbm.at[pl.ds(base, K)], dst_v)

            @pl.when(cid == 0)
            def _():
                pltpu.sync_copy(exd_hbm.at[pl.ds(base, K)], exd_v)
                pltpu.sync_copy(exd_v, acc.at[dst_v], add=True)

            @pl.when(cid == 1)
            def _():
                pltpu.sync_copy(ea_hbm.at[pl.ds(base4, K // 4)], ea_v)

                def edge(e, carry2):
                    eav = ea_v[lax.shift_right_logical(e, 2), pl.ds(0, 16)]
                    gidx = lax.shift_left(jnp.bitwise_and(e, 3), 2) + eaoff
                    cntg = eav.at[gidx].get(mode="promise_in_bounds")
                    stage[e, pl.ds(0, 16)] = cnt1 + m14f * cntg
                    return carry2
                lax.fori_loop(0, K, edge, 0)
                pltpu.sync_copy(stage, acc.at[dst_v], add=True)
            return carry
        lax.fori_loop(0, CHUNKS, chunk, 0)
        plsc.subcore_barrier()

        rows = pl.ds(row0, ROWS_PT)

        @pl.when(cid == 0)
        def _():
            pltpu.sync_copy(acc.at[rows], oden.at[rows])

        @pl.when(cid == 1)
        def _():
            pltpu.sync_copy(acc.at[rows], ocnt.at[rows])

    f32 = jnp.float32
    return pl.kernel(
        body, mesh=mesh,
        compiler_params=pltpu.CompilerParams(use_tc_tiling_on_sc=False),
        out_type=[jax.ShapeDtypeStruct((N_PAD, 16), f32),
                  jax.ShapeDtypeStruct((N_PAD, 16), f32)],
        scratch_types=[
            pltpu.VMEM((K,), jnp.int32),
            pltpu.VMEM((K, 16), f32),
            pltpu.VMEM((K // 4, 16), f32),
            pltpu.VMEM((K, 16), f32),
            pltpu.VMEM_SHARED((N_PAD, 16), f32),
            pltpu.SemaphoreType.DMA,
        ],
    )


_EBLK = 6264
_EGRID = E_PAD // _EBLK


def _eat_proj_body(ea_ref, we1_ref, we2_ref, e1_ref, e2_ref):
    ea = ea_ref[...]
    e1_ref[...] = jnp.dot(ea, we1_ref[...], preferred_element_type=jnp.float32)
    e2_ref[...] = jnp.dot(ea, we2_ref[...], preferred_element_type=jnp.float32)


def _eat_proj(eap, We1, We2):
    full = lambda *s: pl.BlockSpec(s, lambda i: (0,) * len(s))
    we4_1 = jnp.concatenate([We1, jnp.zeros((1, 64), jnp.float32)], axis=0)
    we4_2 = jnp.concatenate([We2, jnp.zeros((1, 64), jnp.float32)], axis=0)
    return pl.pallas_call(
        _eat_proj_body,
        grid=(_EGRID,),
        in_specs=[pl.BlockSpec((_EBLK, 4), lambda i: (i, 0)),
                  full(4, 64), full(4, 64)],
        out_specs=[pl.BlockSpec((_EBLK, 64), lambda i: (i, 0))] * 2,
        out_shape=[jax.ShapeDtypeStruct((E_PAD, 64), jnp.float32)] * 2,
    )(eap, we4_1, we4_2)

def _node_proj_body(x_ref, wl_ref, bl_ref, wr_ref, br_ref, xl_ref, xr_ref):
    x = x_ref[...]
    xl_ref[...] = jnp.dot(x, wl_ref[...], preferred_element_type=jnp.float32) + bl_ref[...]
    xr_ref[...] = jnp.dot(x, wr_ref[...], preferred_element_type=jnp.float32) + br_ref[...]


def _node_proj(x, Wl, bl, Wr, br):
    d_in = x.shape[1]
    d_out = Wl.shape[1]
    full = lambda *s: pl.BlockSpec(s, lambda i: (0,) * len(s))
    return pl.pallas_call(
        _node_proj_body,
        grid=(_GRID,),
        in_specs=[
            pl.BlockSpec((_BLK, d_in), lambda i: (i, 0)),
            full(d_in, d_out), full(1, d_out), full(d_in, d_out), full(1, d_out),
        ],
        out_specs=[pl.BlockSpec((_BLK, d_out), lambda i: (i, 0))] * 2,
        out_shape=[jax.ShapeDtypeStruct((N, d_out), jnp.float32)] * 2,
    )(x, Wl, bl.reshape(1, -1), Wr, br.reshape(1, -1))


def _node_pass1_body(nlo_ref, nhi_ref, den_ref, cnt_ref, xl_ref, xr_ref,
                     we_ref, att_ref, bias_ref, g_ref, be_ref,
                     wl2_ref, bl2_ref, wr2_ref, br2_ref,
                     xl2_ref, xr2_ref):
    nlo = nlo_ref[...]
    nhi = nhi_ref[...]
    den = den_ref[...][:, 0:4]
    cntea = cnt_ref[...][:, 0:4]
    cnt = jnp.maximum(cntea[:, 0:1], 1.0)
    lattr = cntea[:, 1:4] / cnt
    eat = jnp.dot(lattr, we_ref[...], preferred_element_type=jnp.float32)
    xl = xl_ref[...]
    m = xl + xr_ref[...] + eat
    m = jnp.where(m > 0, m, 0.2 * m)
    t = m * att_ref[...]
    outs = []
    for h in range(HEADS):
        sl = slice(h * HC, (h + 1) * HC)
        num_h = nlo[:, (h % 2) * HC:(h % 2 + 1) * HC] if h < 2 else \
            nhi[:, (h % 2) * HC:(h % 2 + 1) * HC]
        alpha = jnp.sum(t[:, sl], axis=1, keepdims=True)
        ex = jnp.exp(alpha)
        numf = num_h + ex * xl[:, sl]
        denf = den[:, h:h + 1] + ex
        outs.append(numf / (denf + 1e-16))
    o = jnp.concatenate(outs, axis=1) + bias_ref[...]
    mu = jnp.mean(o, axis=1, keepdims=True)
    var = jnp.mean((o - mu) ** 2, axis=1, keepdims=True)
    o = (o - mu) * jax.lax.rsqrt(var + 1e-5) * g_ref[...] + be_ref[...]
    o = jnp.where(o > 0, o, jnp.exp(jnp.minimum(o, 0.0)) - 1.0)
    xl2_ref[...] = jnp.dot(o, wl2_ref[...], preferred_element_type=jnp.float32) + bl2_ref[...]
    xr2_ref[...] = jnp.dot(o, wr2_ref[...], preferred_element_type=jnp.float32) + br2_ref[...]


def _node_pass1(nlo, nhi, den16, cnt16, xl1, xr1, We1, att1, bias1, g1, be1,
                Wl2, bl2, Wr2, br2):
    full = lambda *s: pl.BlockSpec(s, lambda i: (0,) * len(s))
    blk64 = pl.BlockSpec((_BLK, 64), lambda i: (i, 0))
    blk32 = pl.BlockSpec((_BLK, 32), lambda i: (i, 0))
    blk16 = pl.BlockSpec((_BLK, 16), lambda i: (i, 0))
    return pl.pallas_call(
        _node_pass1_body,
        grid=(_GRID,),
        in_specs=[
            blk32, blk32, blk16, blk16, blk64, blk64,
            full(3, 64), full(1, 64), full(1, 64), full(1, 64), full(1, 64),
            full(64, 64), full(1, 64), full(64, 64), full(1, 64),
        ],
        out_specs=[blk64, blk64],
        out_shape=[jax.ShapeDtypeStruct((N, 64), jnp.float32)] * 2,
    )(nlo, nhi, den16, cnt16, xl1, xr1,
      We1, att1.reshape(1, 64), bias1.reshape(1, 64), g1.reshape(1, 64),
      be1.reshape(1, 64), Wl2, bl2.reshape(1, 64), Wr2, br2.reshape(1, 64))


def _node_pass2_body(nlo_ref, nhi_ref, den_ref, cnt_ref, xl_ref, xr_ref,
                     we_ref, att_ref, bias_ref, g_ref, be_ref, out_ref):
    cntea = cnt_ref[...][:, 0:4]
    cnt = jnp.maximum(cntea[:, 0:1], 1.0)
    lattr = cntea[:, 1:4] / cnt
    eat = jnp.dot(lattr, we_ref[...], preferred_element_type=jnp.float32)
    xl = xl_ref[...]
    m = xl + xr_ref[...] + eat
    m = jnp.where(m > 0, m, 0.2 * m)
    alpha = jnp.sum(m * att_ref[...], axis=1, keepdims=True)
    ex = jnp.exp(alpha)
    num = jnp.concatenate([nlo_ref[...], nhi_ref[...]], axis=1)
    numf = num + ex * xl
    denf = den_ref[...][:, 0:1] + ex
    o = numf / (denf + 1e-16) + bias_ref[...]
    mu = jnp.mean(o, axis=1, keepdims=True)
    var = jnp.mean((o - mu) ** 2, axis=1, keepdims=True)
    out_ref[...] = (o - mu) * jax.lax.rsqrt(var + 1e-5) * g_ref[...] + be_ref[...]


def _node_pass2(nlo, nhi, den16, cnt16, xl2, xr2, We2, att2, bias2, g2, be2):
    full = lambda *s: pl.BlockSpec(s, lambda i: (0,) * len(s))
    blk64 = pl.BlockSpec((_BLK, 64), lambda i: (i, 0))
    blk32 = pl.BlockSpec((_BLK, 32), lambda i: (i, 0))
    blk16 = pl.BlockSpec((_BLK, 16), lambda i: (i, 0))
    return pl.pallas_call(
        _node_pass2_body,
        grid=(_GRID,),
        in_specs=[
            blk32, blk32, blk16, blk16, blk64, blk64,
            full(3, 64), full(1, 64), full(1, 64), full(1, 64), full(1, 64),
        ],
        out_specs=blk64,
        out_shape=jax.ShapeDtypeStruct((N, 64), jnp.float32),
    )(nlo, nhi, den16, cnt16, xl2, xr2,
      We2, att2.reshape(1, 64), bias2.reshape(1, 64), g2.reshape(1, 64),
      be2.reshape(1, 64))


def kernel(x, edge_attr, edge_index, Wl1, bl1, Wr1, br1, We1, att1, bias1,
           g1, be1, Wl2, bl2, Wr2, br2, We2, att2, bias2, g2, be2):
    src = edge_index[0].astype(jnp.int32)
    dst = edge_index[1].astype(jnp.int32)
    pad = E_PAD - E
    srcp = jnp.concatenate([src, jnp.zeros((pad,), jnp.int32)])
    dstp = jnp.concatenate([dst, jnp.full((pad,), N, jnp.int32)])
    eap = jnp.concatenate(
        [jnp.concatenate([edge_attr,
                          jnp.zeros((E, 1), jnp.float32)], axis=1),
         jnp.zeros((pad, 4), jnp.float32)], axis=0)
    ea4 = eap.reshape(E_PAD // 4, 16)
    zero32 = jnp.zeros((N_PAD, 32), jnp.float32)
    zero16 = jnp.zeros((N_PAD, 16), jnp.float32)

    xl1, xr1 = _node_proj(x, Wl1, bl1, Wr1, br1)
    eat1, eat2 = _eat_proj(eap, We1, We2)

    ep1 = _make_edge_pass(4)
    xp = _make_extra_pass()
    nlo1, nhi1, exd1 = ep1(srcp, dstp, eat1, xl1, xr1, att1.reshape(1, 64),
                           zero32)
    den1, cnt1 = xp(dstp, exd1, ea4, zero16)

    xl2, xr2 = _node_pass1(nlo1, nhi1, den1, cnt1, xl1, xr1, We1, att1,
                           bias1, g1, be1, Wl2, bl2, Wr2, br2)

    ep2 = _make_edge_pass(1)
    nlo2, nhi2, exd2 = ep2(srcp, dstp, eat2, xl2, xr2, att2.reshape(1, 64),
                           zero32)
    den2, cnt2 = xp(dstp, exd2, ea4, zero16)

    return _node_pass2(nlo2, nhi2, den2, cnt2, xl2, xr2, We2, att2,
                       bias2, g2, be2)


# speedup vs baseline: 13.0844x; 1.1816x over previous
import functools

import numpy as np

import jax
import jax.numpy as jnp
from jax import lax
from jax.experimental import pallas as pl
from jax.experimental.pallas import tpu as pltpu
from jax.experimental.pallas import tpu_sc as plsc

N = 50000
E = 800000
HEADS = 4
HC = 16

_BLK = 2000
_GRID = N // _BLK

K = 96
NTILES = 16
CHUNKS = 522
EPT = K * CHUNKS
E_PAD = EPT * NTILES
ROWS_PT = 3128
N_PAD = ROWS_PT * NTILES


def _make_edge_pass(heads):
    mesh = plsc.VectorSubcoreMesh(core_axis_name="c", subcore_axis_name="s",
                                  num_cores=2, num_subcores=16)

    def body(src_hbm, dst_hbm, ea_hbm, xl_hbm, xr_hbm, const_hbm,
             zero_hbm,
             onum0, onum1, exd_hbm,
             src_v0, dst_v0, ea_v0, xlg0, xrg0,
             src_v1, dst_v1, ea_v1, xlg1, xrg1,
             consts_v, stage, stage_exd, acc, sem0, sem1):
        cid = lax.axis_index("c")
        sid = lax.axis_index("s")
        row0 = pl.multiple_of(sid * ROWS_PT, 64)

        pltpu.sync_copy(zero_hbm.at[pl.ds(row0, ROWS_PT)],
                        acc.at[pl.ds(row0, ROWS_PT)])
        pltpu.sync_copy(const_hbm, consts_v)

        we = [[consts_v[k, pl.ds(h * 16, 16)] for h in range(4)]
              for k in range(3)]
        att = [consts_v[3, pl.ds(h * 16, 16)] for h in range(4)]
        lanes = lax.iota(jnp.int32, 16)
        lf = lax.convert_element_type(lanes, jnp.float32)
        clamp01 = lambda v: jnp.minimum(jnp.maximum(v, 0.0), 1.0)
        oh = [clamp01(1.0 - jnp.abs(lf - float(i))) for i in range(4)]
        c0f = lax.broadcast(
            jnp.float32(1.0) - lax.convert_element_type(cid, jnp.float32),
            (16,))
        c1f = 1.0 - c0f
        ebase = sid * EPT
        shuf = [lanes ^ s for s in (8, 4, 2, 1)]
        bufs = [(src_v0, dst_v0, ea_v0, xlg0, xrg0, sem0),
                (src_v1, dst_v1, ea_v1, xlg1, xrg1, sem1)]

        def allsum(v):
            for sx in shuf:
                v = v + v.at[sx].get(mode="promise_in_bounds")
            return v

        def ebase_of(ch):
            return pl.multiple_of(ebase + ch * K, 32)

        def issue(ch, b):
            s, d, ea, xg, rg, sem = bufs[b]
            base = ebase_of(ch)
            base4 = pl.multiple_of((ebase + ch * K) // 4, 8)
            pltpu.sync_copy(src_hbm.at[pl.ds(base, K)], s)
            pltpu.sync_copy(dst_hbm.at[pl.ds(base, K)], d)
            pltpu.sync_copy(ea_hbm.at[pl.ds(base4, K // 4)], ea)
            pltpu.async_copy(xl_hbm.at[s], xg, sem)
            pltpu.async_copy(xr_hbm.at[d], rg, sem)

        def compute(ch, b):
            s, d, ea, xg, rg, sem = bufs[b]
            pltpu.make_async_copy(xl_hbm.at[s], xg, sem).wait()
            pltpu.make_async_copy(xr_hbm.at[d], rg, sem).wait()

            def edge(e, carry2):
                eav = ea[lax.shift_right_logical(e, 2), pl.ds(0, 16)]
                e4 = lax.shift_left(jnp.bitwise_and(e, 3), 2)
                eab = [eav.at[lax.broadcast(e4 + k, (16,))].get(
                    mode="promise_in_bounds") for k in range(3)]
                xs = []
                exs = []
                tacc = None
                for h in range(4):
                    xlh = xg[e, pl.ds(h * 16, 16)]
                    xrh = rg[e, pl.ds(h * 16, 16)]
                    m = (xlh + xrh + eab[0] * we[0][h] + eab[1] * we[1][h]
                         + eab[2] * we[2][h])
                    m = jnp.maximum(m, 0.0) + 0.2 * jnp.minimum(m, 0.0)
                    t = m * att[h]
                    xs.append(xlh)
                    if heads == 4:
                        exs.append(jnp.exp(allsum(t)))
                    else:
                        tacc = t if tacc is None else tacc + t
                if heads == 1:
                    ex = jnp.exp(allsum(tacc))
                    exs = [ex, ex, ex, ex]
                na = c0f * (exs[0] * xs[0]) + c1f * (exs[2] * xs[2])
                nb = c0f * (exs[1] * xs[1]) + c1f * (exs[3] * xs[3])
                if heads == 4:
                    exrow = (oh[0] * exs[0] + oh[1] * exs[1]
                             + oh[2] * exs[2] + oh[3] * exs[3])
                else:
                    exrow = oh[0] * exs[0]
                stage[e, pl.ds(0, 16)] = na
                stage[e, pl.ds(16, 16)] = nb
                stage_exd[e, pl.ds(0, 16)] = exrow
                return carry2
            lax.fori_loop(0, K, edge, 0)

            pltpu.sync_copy(stage, acc.at[d], add=True)

            @pl.when(cid == 0)
            def _():
                pltpu.sync_copy(stage_exd, exd_hbm.at[pl.ds(ebase_of(ch), K)])

        plsc.subcore_barrier()

        issue(0, 0)

        def pair(g, carry):
            ch0 = 2 * g
            issue(ch0 + 1, 1)
            compute(ch0, 0)

            @pl.when(ch0 + 2 < CHUNKS)
            def _():
                issue(ch0 + 2, 0)
            compute(ch0 + 1, 1)
            return carry
        lax.fori_loop(0, CHUNKS // 2, pair, 0)
        plsc.subcore_barrier()

        rows = pl.ds(row0, ROWS_PT)

        @pl.when(cid == 0)
        def _():
            pltpu.sync_copy(acc.at[rows], onum0.at[rows])

        @pl.when(cid == 1)
        def _():
            pltpu.sync_copy(acc.at[rows], onum1.at[rows])

    f32 = jnp.float32
    dbuf = [
        pltpu.VMEM((K,), jnp.int32),
        pltpu.VMEM((K,), jnp.int32),
        pltpu.VMEM((K // 4, 16), f32),
        pltpu.VMEM((K, 64), f32),
        pltpu.VMEM((K, 64), f32),
    ]
    return pl.kernel(
        body, mesh=mesh,
        compiler_params=pltpu.CompilerParams(use_tc_tiling_on_sc=False),
        out_type=[jax.ShapeDtypeStruct((N_PAD, 32), f32),
                  jax.ShapeDtypeStruct((N_PAD, 32), f32),
                  jax.ShapeDtypeStruct((E_PAD, 16), f32)],
        scratch_types=dbuf + dbuf + [
            pltpu.VMEM((4, 64), f32),
            pltpu.VMEM((K, 32), f32),
            pltpu.VMEM((K, 16), f32),
            pltpu.VMEM_SHARED((N_PAD, 32), f32),
            pltpu.SemaphoreType.DMA,
            pltpu.SemaphoreType.DMA,
        ],
    )


def _make_extra_pass(do_cnt):
    mesh = plsc.VectorSubcoreMesh(core_axis_name="c", subcore_axis_name="s",
                                  num_cores=2, num_subcores=16)

    def body(dst_hbm, exd_hbm, ea_hbm, zero_hbm,
             oden, ocnt,
             dst_v, exd_v, ea_v, stage, acc, sem):
        cid = lax.axis_index("c")
        sid = lax.axis_index("s")
        row0 = pl.multiple_of(sid * ROWS_PT, 64)

        pltpu.sync_copy(zero_hbm.at[pl.ds(row0, ROWS_PT)],
                        acc.at[pl.ds(row0, ROWS_PT)])

        lanes = lax.iota(jnp.int32, 16)
        lf = lax.convert_element_type(lanes, jnp.float32)
        clamp01 = lambda v: jnp.minimum(jnp.maximum(v, 0.0), 1.0)
        cnt1 = clamp01(1.0 - lf)
        eaoff = jnp.minimum(jnp.maximum(lanes - 1, 0), 3)
        m14f = clamp01(lf) * clamp01(4.0 - lf)
        ebase = sid * EPT

        plsc.subcore_barrier()

        def chunk(ch, carry):
            base = pl.multiple_of(ebase + ch * K, 32)
            base4 = pl.multiple_of((ebase + ch * K) // 4, 8)
            pltpu.sync_copy(dst_hbm.at[pl.ds(base, K)], dst_v)

            @pl.when(cid == 0)
            def _():
                pltpu.sync_copy(exd_hbm.at[pl.ds(base, K)], exd_v)
                pltpu.sync_copy(exd_v, acc.at[dst_v], add=True)

            if do_cnt:
                @pl.when(cid == 1)
                def _():
                    pltpu.sync_copy(ea_hbm.at[pl.ds(base4, K // 4)], ea_v)

                    def edge(e, carry2):
                        eav = ea_v[lax.shift_right_logical(e, 2),
                                   pl.ds(0, 16)]
                        gidx = lax.shift_left(jnp.bitwise_and(e, 3), 2) + eaoff
                        cntg = eav.at[gidx].get(mode="promise_in_bounds")
                        stage[e, pl.ds(0, 16)] = cnt1 + m14f * cntg
                        return carry2
                    lax.fori_loop(0, K, edge, 0)
                    pltpu.sync_copy(stage, acc.at[dst_v], add=True)
            return carry
        lax.fori_loop(0, CHUNKS, chunk, 0)
        plsc.subcore_barrier()

        rows = pl.ds(row0, ROWS_PT)

        @pl.when(cid == 0)
        def _():
            pltpu.sync_copy(acc.at[rows], oden.at[rows])

        @pl.when(cid == 1)
        def _():
            pltpu.sync_copy(acc.at[rows], ocnt.at[rows])

    f32 = jnp.float32
    return pl.kernel(
        body, mesh=mesh,
        compiler_params=pltpu.CompilerParams(use_tc_tiling_on_sc=False),
        out_type=[jax.ShapeDtypeStruct((N_PAD, 16), f32),
                  jax.ShapeDtypeStruct((N_PAD, 16), f32)],
        scratch_types=[
            pltpu.VMEM((K,), jnp.int32),
            pltpu.VMEM((K, 16), f32),
            pltpu.VMEM((K // 4, 16), f32),
            pltpu.VMEM((K, 16), f32),
            pltpu.VMEM_SHARED((N_PAD, 16), f32),
            pltpu.SemaphoreType.DMA,
        ],
    )


_EBLK = 6264
_EGRID = E_PAD // _EBLK


def _eat_proj_body(ea_ref, we1_ref, we2_ref, e1_ref, e2_ref):
    ea = ea_ref[...]
    e1_ref[...] = jnp.dot(ea, we1_ref[...], preferred_element_type=jnp.float32)
    e2_ref[...] = jnp.dot(ea, we2_ref[...], preferred_element_type=jnp.float32)


def _eat_proj(eap, We1, We2):
    full = lambda *s: pl.BlockSpec(s, lambda i: (0,) * len(s))
    we4_1 = jnp.concatenate([We1, jnp.zeros((1, 64), jnp.float32)], axis=0)
    we4_2 = jnp.concatenate([We2, jnp.zeros((1, 64), jnp.float32)], axis=0)
    return pl.pallas_call(
        _eat_proj_body,
        grid=(_EGRID,),
        in_specs=[pl.BlockSpec((_EBLK, 4), lambda i: (i, 0)),
                  full(4, 64), full(4, 64)],
        out_specs=[pl.BlockSpec((_EBLK, 64), lambda i: (i, 0))] * 2,
        out_shape=[jax.ShapeDtypeStruct((E_PAD, 64), jnp.float32)] * 2,
    )(eap, we4_1, we4_2)

def _node_proj_body(x_ref, wl_ref, bl_ref, wr_ref, br_ref, xl_ref, xr_ref):
    x = x_ref[...]
    xl_ref[...] = jnp.dot(x, wl_ref[...], preferred_element_type=jnp.float32) + bl_ref[...]
    xr_ref[...] = jnp.dot(x, wr_ref[...], preferred_element_type=jnp.float32) + br_ref[...]


def _node_proj(x, Wl, bl, Wr, br):
    d_in = x.shape[1]
    d_out = Wl.shape[1]
    full = lambda *s: pl.BlockSpec(s, lambda i: (0,) * len(s))
    return pl.pallas_call(
        _node_proj_body,
        grid=(_GRID,),
        in_specs=[
            pl.BlockSpec((_BLK, d_in), lambda i: (i, 0)),
            full(d_in, d_out), full(1, d_out), full(d_in, d_out), full(1, d_out),
        ],
        out_specs=[pl.BlockSpec((_BLK, d_out), lambda i: (i, 0))] * 2,
        out_shape=[jax.ShapeDtypeStruct((N, d_out), jnp.float32)] * 2,
    )(x, Wl, bl.reshape(1, -1), Wr, br.reshape(1, -1))


def _node_pass1_body(nlo_ref, nhi_ref, den_ref, cnt_ref, xl_ref, xr_ref,
                     we_ref, att_ref, bias_ref, g_ref, be_ref,
                     wl2_ref, bl2_ref, wr2_ref, br2_ref,
                     xl2_ref, xr2_ref):
    nlo = nlo_ref[...]
    nhi = nhi_ref[...]
    den = den_ref[...][:, 0:4]
    cntea = cnt_ref[...][:, 0:4]
    cnt = jnp.maximum(cntea[:, 0:1], 1.0)
    lattr = cntea[:, 1:4] / cnt
    eat = jnp.dot(lattr, we_ref[...], preferred_element_type=jnp.float32)
    xl = xl_ref[...]
    m = xl + xr_ref[...] + eat
    m = jnp.where(m > 0, m, 0.2 * m)
    t = m * att_ref[...]
    outs = []
    for h in range(HEADS):
        sl = slice(h * HC, (h + 1) * HC)
        num_h = nlo[:, (h % 2) * HC:(h % 2 + 1) * HC] if h < 2 else \
            nhi[:, (h % 2) * HC:(h % 2 + 1) * HC]
        alpha = jnp.sum(t[:, sl], axis=1, keepdims=True)
        ex = jnp.exp(alpha)
        numf = num_h + ex * xl[:, sl]
        denf = den[:, h:h + 1] + ex
        outs.append(numf / (denf + 1e-16))
    o = jnp.concatenate(outs, axis=1) + bias_ref[...]
    mu = jnp.mean(o, axis=1, keepdims=True)
    var = jnp.mean((o - mu) ** 2, axis=1, keepdims=True)
    o = (o - mu) * jax.lax.rsqrt(var + 1e-5) * g_ref[...] + be_ref[...]
    o = jnp.where(o > 0, o, jnp.exp(jnp.minimum(o, 0.0)) - 1.0)
    xl2_ref[...] = jnp.dot(o, wl2_ref[...], preferred_element_type=jnp.float32) + bl2_ref[...]
    xr2_ref[...] = jnp.dot(o, wr2_ref[...], preferred_element_type=jnp.float32) + br2_ref[...]


def _node_pass1(nlo, nhi, den16, cnt16, xl1, xr1, We1, att1, bias1, g1, be1,
                Wl2, bl2, Wr2, br2):
    full = lambda *s: pl.BlockSpec(s, lambda i: (0,) * len(s))
    blk64 = pl.BlockSpec((_BLK, 64), lambda i: (i, 0))
    blk32 = pl.BlockSpec((_BLK, 32), lambda i: (i, 0))
    blk16 = pl.BlockSpec((_BLK, 16), lambda i: (i, 0))
    return pl.pallas_call(
        _node_pass1_body,
        grid=(_GRID,),
        in_specs=[
            blk32, blk32, blk16, blk16, blk64, blk64,
            full(3, 64), full(1, 64), full(1, 64), full(1, 64), full(1, 64),
            full(64, 64), full(1, 64), full(64, 64), full(1, 64),
        ],
        out_specs=[blk64, blk64],
        out_shape=[jax.ShapeDtypeStruct((N, 64), jnp.float32)] * 2,
    )(nlo, nhi, den16, cnt16, xl1, xr1,
      We1, att1.reshape(1, 64), bias1.reshape(1, 64), g1.reshape(1, 64),
      be1.reshape(1, 64), Wl2, bl2.reshape(1, 64), Wr2, br2.reshape(1, 64))


def _node_pass2_body(nlo_ref, nhi_ref, den_ref, cnt_ref, xl_ref, xr_ref,
                     we_ref, att_ref, bias_ref, g_ref, be_ref, out_ref):
    cntea = cnt_ref[...][:, 0:4]
    cnt = jnp.maximum(cntea[:, 0:1], 1.0)
    lattr = cntea[:, 1:4] / cnt
    eat = jnp.dot(lattr, we_ref[...], preferred_element_type=jnp.float32)
    xl = xl_ref[...]
    m = xl + xr_ref[...] + eat
    m = jnp.where(m > 0, m, 0.2 * m)
    alpha = jnp.sum(m * att_ref[...], axis=1, keepdims=True)
    ex = jnp.exp(alpha)
    num = jnp.concatenate([nlo_ref[...], nhi_ref[...]], axis=1)
    numf = num + ex * xl
    denf = den_ref[...][:, 0:1] + ex
    o = numf / (denf + 1e-16) + bias_ref[...]
    mu = jnp.mean(o, axis=1, keepdims=True)
    var = jnp.mean((o - mu) ** 2, axis=1, keepdims=True)
    out_ref[...] = (o - mu) * jax.lax.rsqrt(var + 1e-5) * g_ref[...] + be_ref[...]


def _node_pass2(nlo, nhi, den16, cnt16, xl2, xr2, We2, att2, bias2, g2, be2):
    full = lambda *s: pl.BlockSpec(s, lambda i: (0,) * len(s))
    blk64 = pl.BlockSpec((_BLK, 64), lambda i: (i, 0))
    blk32 = pl.BlockSpec((_BLK, 32), lambda i: (i, 0))
    blk16 = pl.BlockSpec((_BLK, 16), lambda i: (i, 0))
    return pl.pallas_call(
        _node_pass2_body,
        grid=(_GRID,),
        in_specs=[
            blk32, blk32, blk16, blk16, blk64, blk64,
            full(3, 64), full(1, 64), full(1, 64), full(1, 64), full(1, 64),
        ],
        out_specs=blk64,
        out_shape=jax.ShapeDtypeStruct((N, 64), jnp.float32),
    )(nlo, nhi, den16, cnt16, xl2, xr2,
      We2, att2.reshape(1, 64), bias2.reshape(1, 64), g2.reshape(1, 64),
      be2.reshape(1, 64))


def kernel(x, edge_attr, edge_index, Wl1, bl1, Wr1, br1, We1, att1, bias1,
           g1, be1, Wl2, bl2, Wr2, br2, We2, att2, bias2, g2, be2):
    src = edge_index[0].astype(jnp.int32)
    dst = edge_index[1].astype(jnp.int32)
    pad = E_PAD - E
    srcp = jnp.concatenate([src, jnp.zeros((pad,), jnp.int32)])
    dstp = jnp.concatenate([dst, jnp.full((pad,), N, jnp.int32)])
    eap = jnp.concatenate(
        [jnp.concatenate([edge_attr,
                          jnp.zeros((E, 1), jnp.float32)], axis=1),
         jnp.zeros((pad, 4), jnp.float32)], axis=0)
    ea4 = eap.reshape(E_PAD // 4, 16)
    zero32 = jnp.zeros((N_PAD, 32), jnp.float32)
    zero16 = jnp.zeros((N_PAD, 16), jnp.float32)

    xl1, xr1 = _node_proj(x, Wl1, bl1, Wr1, br1)

    const1 = jnp.concatenate([We1, att1.reshape(1, 64)], axis=0)
    const2 = jnp.concatenate([We2, att2.reshape(1, 64)], axis=0)

    ep1 = _make_edge_pass(4)
    nlo1, nhi1, exd1 = ep1(srcp, dstp, ea4, xl1, xr1, const1, zero32)
    den1, cnt1 = _make_extra_pass(True)(dstp, exd1, ea4, zero16)

    xl2, xr2 = _node_pass1(nlo1, nhi1, den1, cnt1, xl1, xr1, We1, att1,
                           bias1, g1, be1, Wl2, bl2, Wr2, br2)

    ep2 = _make_edge_pass(1)
    nlo2, nhi2, exd2 = ep2(srcp, dstp, ea4, xl2, xr2, const2, zero32)
    den2, cnt2 = _make_extra_pass(False)(dstp, exd2, ea4, zero16)

    return _node_pass2(nlo2, nhi2, den2, cnt2, xl2, xr2, We2, att2,
                       bias2, g2, be2)


# speedup vs baseline: 13.1045x; 1.0015x over previous
import functools

import numpy as np

import jax
import jax.numpy as jnp
from jax import lax
from jax.experimental import pallas as pl
from jax.experimental.pallas import tpu as pltpu
from jax.experimental.pallas import tpu_sc as plsc

N = 50000
E = 800000
HEADS = 4
HC = 16

_BLK = 2000
_GRID = N // _BLK

K = 96
NTILES = 16
CHUNKS = 522
EPT = K * CHUNKS
E_PAD = EPT * NTILES
ROWS_PT = 3128
N_PAD = ROWS_PT * NTILES


def _make_edge_pass(heads):
    mesh = plsc.VectorSubcoreMesh(core_axis_name="c", subcore_axis_name="s",
                                  num_cores=2, num_subcores=16)

    def body(src_hbm, dst_hbm, ea_hbm, xl_hbm, xr_hbm, const_hbm,
             zero_hbm,
             onum0, onum1, exd_hbm,
             src_v0, dst_v0, ea_v0, xlg0, xrg0,
             src_v1, dst_v1, ea_v1, xlg1, xrg1,
             consts_v, stage, stage_exd, acc, sem0, sem1):
        cid = lax.axis_index("c")
        sid = lax.axis_index("s")
        row0 = pl.multiple_of(sid * ROWS_PT, 64)

        pltpu.sync_copy(zero_hbm.at[pl.ds(row0, ROWS_PT)],
                        acc.at[pl.ds(row0, ROWS_PT)])
        pltpu.sync_copy(const_hbm, consts_v)

        we = [[consts_v[k, pl.ds(h * 16, 16)] for h in range(4)]
              for k in range(3)]
        att = [consts_v[3, pl.ds(h * 16, 16)] for h in range(4)]
        lanes = lax.iota(jnp.int32, 16)
        lf = lax.convert_element_type(lanes, jnp.float32)
        clamp01 = lambda v: jnp.minimum(jnp.maximum(v, 0.0), 1.0)
        oh = [clamp01(1.0 - jnp.abs(lf - float(i))) for i in range(4)]
        c0f = lax.broadcast(
            jnp.float32(1.0) - lax.convert_element_type(cid, jnp.float32),
            (16,))
        c1f = 1.0 - c0f
        ebase = sid * EPT
        shuf = [lanes ^ s for s in (8, 4, 2, 1)]
        bufs = [(src_v0, dst_v0, ea_v0, xlg0, xrg0, sem0),
                (src_v1, dst_v1, ea_v1, xlg1, xrg1, sem1)]

        def allsum(v):
            for sx in shuf:
                v = v + v.at[sx].get(mode="promise_in_bounds")
            return v

        def ebase_of(ch):
            return pl.multiple_of(ebase + ch * K, 32)

        def issue(ch, b):
            s, d, ea, xg, rg, sem = bufs[b]
            base = ebase_of(ch)
            base4 = pl.multiple_of((ebase + ch * K) // 4, 8)
            pltpu.sync_copy(src_hbm.at[pl.ds(base, K)], s)
            pltpu.sync_copy(dst_hbm.at[pl.ds(base, K)], d)
            pltpu.sync_copy(ea_hbm.at[pl.ds(base4, K // 4)], ea)
            pltpu.async_copy(xl_hbm.at[s], xg, sem)
            pltpu.async_copy(xr_hbm.at[d], rg, sem)

        def compute(ch, b):
            s, d, ea, xg, rg, sem = bufs[b]
            pltpu.make_async_copy(xl_hbm.at[s], xg, sem).wait()
            pltpu.make_async_copy(xr_hbm.at[d], rg, sem).wait()

            def edge(e, carry2):
                eav = ea[lax.shift_right_logical(e, 2), pl.ds(0, 16)]
                e4 = lax.shift_left(jnp.bitwise_and(e, 3), 2)
                eab = [eav.at[lax.broadcast(e4 + k, (16,))].get(
                    mode="promise_in_bounds") for k in range(3)]
                xs = []
                exs = []
                tacc = None
                for h in range(4):
                    xlh = xg[e, pl.ds(h * 16, 16)]
                    xrh = rg[e, pl.ds(h * 16, 16)]
                    m = (xlh + xrh + eab[0] * we[0][h] + eab[1] * we[1][h]
                         + eab[2] * we[2][h])
                    m = jnp.maximum(m, 0.0) + 0.2 * jnp.minimum(m, 0.0)
                    t = m * att[h]
                    xs.append(xlh)
                    if heads == 4:
                        exs.append(jnp.exp(allsum(t)))
                    else:
                        tacc = t if tacc is None else tacc + t
                if heads == 1:
                    ex = jnp.exp(allsum(tacc))
                    exs = [ex, ex, ex, ex]
                na = c0f * (exs[0] * xs[0]) + c1f * (exs[2] * xs[2])
                nb = c0f * (exs[1] * xs[1]) + c1f * (exs[3] * xs[3])
                if heads == 4:
                    exrow = (oh[0] * exs[0] + oh[1] * exs[1]
                             + oh[2] * exs[2] + oh[3] * exs[3])
                else:
                    exrow = oh[0] * exs[0]
                stage[e, pl.ds(0, 16)] = na
                stage[e, pl.ds(16, 16)] = nb
                stage_exd[e, pl.ds(0, 16)] = exrow
                return carry2
            lax.fori_loop(0, K, edge, 0)

            pltpu.sync_copy(stage, acc.at[d], add=True)

            @pl.when(cid == 0)
            def _():
                pltpu.sync_copy(stage_exd, exd_hbm.at[pl.ds(ebase_of(ch), K)])

        plsc.subcore_barrier()

        issue(0, 0)

        def pair(g, carry):
            ch0 = 2 * g
            issue(ch0 + 1, 1)
            compute(ch0, 0)

            @pl.when(ch0 + 2 < CHUNKS)
            def _():
                issue(ch0 + 2, 0)
            compute(ch0 + 1, 1)
            return carry
        lax.fori_loop(0, CHUNKS // 2, pair, 0)
        plsc.subcore_barrier()

        rows = pl.ds(row0, ROWS_PT)

        @pl.when(cid == 0)
        def _():
            pltpu.sync_copy(acc.at[rows], onum0.at[rows])

        @pl.when(cid == 1)
        def _():
            pltpu.sync_copy(acc.at[rows], onum1.at[rows])

    f32 = jnp.float32
    dbuf = [
        pltpu.VMEM((K,), jnp.int32),
        pltpu.VMEM((K,), jnp.int32),
        pltpu.VMEM((K // 4, 16), f32),
        pltpu.VMEM((K, 64), f32),
        pltpu.VMEM((K, 64), f32),
    ]
    return pl.kernel(
        body, mesh=mesh,
        compiler_params=pltpu.CompilerParams(use_tc_tiling_on_sc=False),
        out_type=[jax.ShapeDtypeStruct((N_PAD, 32), f32),
                  jax.ShapeDtypeStruct((N_PAD, 32), f32),
                  jax.ShapeDtypeStruct((E_PAD, 16), f32)],
        scratch_types=dbuf + dbuf + [
            pltpu.VMEM((4, 64), f32),
            pltpu.VMEM((K, 32), f32),
            pltpu.VMEM((K, 16), f32),
            pltpu.VMEM_SHARED((N_PAD, 32), f32),
            pltpu.SemaphoreType.DMA,
            pltpu.SemaphoreType.DMA,
        ],
    )


def _make_extra_pass(do_cnt):
    mesh = plsc.VectorSubcoreMesh(core_axis_name="c", subcore_axis_name="s",
                                  num_cores=2, num_subcores=16)

    def body(dst_hbm, exd_hbm, ea_hbm, zero_hbm,
             oden, ocnt,
             dst_v, exd_v, ea_v, stage, acc, sem):
        cid = lax.axis_index("c")
        sid = lax.axis_index("s")
        row0 = pl.multiple_of(sid * ROWS_PT, 64)

        pltpu.sync_copy(zero_hbm.at[pl.ds(row0, ROWS_PT)],
                        acc.at[pl.ds(row0, ROWS_PT)])

        lanes = lax.iota(jnp.int32, 16)
        lf = lax.convert_element_type(lanes, jnp.float32)
        clamp01 = lambda v: jnp.minimum(jnp.maximum(v, 0.0), 1.0)
        cnt1 = clamp01(1.0 - lf)
        eaoff = jnp.minimum(jnp.maximum(lanes - 1, 0), 3)
        m14f = clamp01(lf) * clamp01(4.0 - lf)
        ebase = sid * EPT

        plsc.subcore_barrier()

        def chunk(ch, carry):
            base = pl.multiple_of(ebase + ch * K, 32)
            base4 = pl.multiple_of((ebase + ch * K) // 4, 8)
            pltpu.sync_copy(dst_hbm.at[pl.ds(base, K)], dst_v)

            @pl.when(cid == 0)
            def _():
                pltpu.sync_copy(exd_hbm.at[pl.ds(base, K)], exd_v)
                pltpu.sync_copy(exd_v, acc.at[dst_v], add=True)

            if do_cnt:
                @pl.when(cid == 1)
                def _():
                    pltpu.sync_copy(ea_hbm.at[pl.ds(base4, K // 4)], ea_v)

                    def edge(e, carry2):
                        eav = ea_v[lax.shift_right_logical(e, 2),
                                   pl.ds(0, 16)]
                        gidx = lax.shift_left(jnp.bitwise_and(e, 3), 2) + eaoff
                        cntg = eav.at[gidx].get(mode="promise_in_bounds")
                        stage[e, pl.ds(0, 16)] = cnt1 + m14f * cntg
                        return carry2
                    lax.fori_loop(0, K, edge, 0)
                    pltpu.sync_copy(stage, acc.at[dst_v], add=True)
            return carry
        lax.fori_loop(0, CHUNKS, chunk, 0)
        plsc.subcore_barrier()

        rows = pl.ds(row0, ROWS_PT)

        @pl.when(cid == 0)
        def _():
            pltpu.sync_copy(acc.at[rows], oden.at[rows])

        @pl.when(cid == 1)
        def _():
            pltpu.sync_copy(acc.at[rows], ocnt.at[rows])

    f32 = jnp.float32
    return pl.kernel(
        body, mesh=mesh,
        compiler_params=pltpu.CompilerParams(use_tc_tiling_on_sc=False),
        out_type=[jax.ShapeDtypeStruct((N_PAD, 16), f32),
                  jax.ShapeDtypeStruct((N_PAD, 16), f32)],
        scratch_types=[
            pltpu.VMEM((K,), jnp.int32),
            pltpu.VMEM((K, 16), f32),
            pltpu.VMEM((K // 4, 16), f32),
            pltpu.VMEM((K, 16), f32),
            pltpu.VMEM_SHARED((N_PAD, 16), f32),
            pltpu.SemaphoreType.DMA,
        ],
    )


_EBLK = 6264
_EGRID = E_PAD // _EBLK


def _eat_proj_body(ea_ref, we1_ref, we2_ref, e1_ref, e2_ref):
    ea = ea_ref[...]
    e1_ref[...] = jnp.dot(ea, we1_ref[...], preferred_element_type=jnp.float32)
    e2_ref[...] = jnp.dot(ea, we2_ref[...], preferred_element_type=jnp.float32)


def _eat_proj(eap, We1, We2):
    full = lambda *s: pl.BlockSpec(s, lambda i: (0,) * len(s))
    we4_1 = jnp.concatenate([We1, jnp.zeros((1, 64), jnp.float32)], axis=0)
    we4_2 = jnp.concatenate([We2, jnp.zeros((1, 64), jnp.float32)], axis=0)
    return pl.pallas_call(
        _eat_proj_body,
        grid=(_EGRID,),
        in_specs=[pl.BlockSpec((_EBLK, 4), lambda i: (i, 0)),
                  full(4, 64), full(4, 64)],
        out_specs=[pl.BlockSpec((_EBLK, 64), lambda i: (i, 0))] * 2,
        out_shape=[jax.ShapeDtypeStruct((E_PAD, 64), jnp.float32)] * 2,
    )(eap, we4_1, we4_2)

def _node_proj_body(x_ref, wl_ref, bl_ref, wr_ref, br_ref, xl_ref, xr_ref):
    x = x_ref[...]
    xl_ref[...] = jnp.dot(x, wl_ref[...], preferred_element_type=jnp.float32) + bl_ref[...]
    xr_ref[...] = jnp.dot(x, wr_ref[...], preferred_element_type=jnp.float32) + br_ref[...]


def _node_proj(x, Wl, bl, Wr, br):
    d_in = x.shape[1]
    d_out = Wl.shape[1]
    full = lambda *s: pl.BlockSpec(s, lambda i: (0,) * len(s))
    return pl.pallas_call(
        _node_proj_body,
        grid=(_GRID,),
        in_specs=[
            pl.BlockSpec((_BLK, d_in), lambda i: (i, 0)),
            full(d_in, d_out), full(1, d_out), full(d_in, d_out), full(1, d_out),
        ],
        out_specs=[pl.BlockSpec((_BLK, d_out), lambda i: (i, 0))] * 2,
        out_shape=[jax.ShapeDtypeStruct((N, d_out), jnp.float32)] * 2,
    )(x, Wl, bl.reshape(1, -1), Wr, br.reshape(1, -1))


def _node_pass1_body(nlo_ref, nhi_ref, den_ref, cnt_ref, xl_ref, xr_ref,
                     we_ref, att_ref, bias_ref, g_ref, be_ref,
                     wl2_ref, bl2_ref, wr2_ref, br2_ref,
                     xl2_ref, xr2_ref):
    nlo = nlo_ref[...]
    nhi = nhi_ref[...]
    den = den_ref[...][:, 0:4]
    cntea = cnt_ref[...][:, 0:4]
    cnt = jnp.maximum(cntea[:, 0:1], 1.0)
    lattr = cntea[:, 1:4] / cnt
    eat = jnp.dot(lattr, we_ref[...], preferred_element_type=jnp.float32)
    xl = xl_ref[...]
    m = xl + xr_ref[...] + eat
    m = jnp.where(m > 0, m, 0.2 * m)
    t = m * att_ref[...]
    outs = []
    for h in range(HEADS):
        sl = slice(h * HC, (h + 1) * HC)
        num_h = nlo[:, (h % 2) * HC:(h % 2 + 1) * HC] if h < 2 else \
            nhi[:, (h % 2) * HC:(h % 2 + 1) * HC]
        alpha = jnp.sum(t[:, sl], axis=1, keepdims=True)
        ex = jnp.exp(alpha)
        numf = num_h + ex * xl[:, sl]
        denf = den[:, h:h + 1] + ex
        outs.append(numf / (denf + 1e-16))
    o = jnp.concatenate(outs, axis=1) + bias_ref[...]
    mu = jnp.mean(o, axis=1, keepdims=True)
    var = jnp.mean((o - mu) ** 2, axis=1, keepdims=True)
    o = (o - mu) * jax.lax.rsqrt(var + 1e-5) * g_ref[...] + be_ref[...]
    o = jnp.where(o > 0, o, jnp.exp(jnp.minimum(o, 0.0)) - 1.0)
    xl2_ref[...] = jnp.dot(o, wl2_ref[...], preferred_element_type=jnp.float32) + bl2_ref[...]
    xr2_ref[...] = jnp.dot(o, wr2_ref[...], preferred_element_type=jnp.float32) + br2_ref[...]


def _node_pass1(nlo, nhi, den16, cnt16, xl1, xr1, We1, att1, bias1, g1, be1,
                Wl2, bl2, Wr2, br2):
    full = lambda *s: pl.BlockSpec(s, lambda i: (0,) * len(s))
    blk64 = pl.BlockSpec((_BLK, 64), lambda i: (i, 0))
    blk32 = pl.BlockSpec((_BLK, 32), lambda i: (i, 0))
    blk16 = pl.BlockSpec((_BLK, 16), lambda i: (i, 0))
    return pl.pallas_call(
        _node_pass1_body,
        grid=(_GRID,),
        in_specs=[
            blk32, blk32, blk16, blk16, blk64, blk64,
            full(3, 64), full(1, 64), full(1, 64), full(1, 64), full(1, 64),
            full(64, 64), full(1, 64), full(64, 64), full(1, 64),
        ],
        out_specs=[blk64, blk64],
        out_shape=[jax.ShapeDtypeStruct((N, 64), jnp.float32)] * 2,
    )(nlo, nhi, den16, cnt16, xl1, xr1,
      We1, att1.reshape(1, 64), bias1.reshape(1, 64), g1.reshape(1, 64),
      be1.reshape(1, 64), Wl2, bl2.reshape(1, 64), Wr2, br2.reshape(1, 64))


def _node_pass2_body(nlo_ref, nhi_ref, den_ref, cnt_ref, xl_ref, xr_ref,
                     we_ref, att_ref, bias_ref, g_ref, be_ref, out_ref):
    cntea = cnt_ref[...][:, 0:4]
    cnt = jnp.maximum(cntea[:, 0:1], 1.0)
    lattr = cntea[:, 1:4] / cnt
    eat = jnp.dot(lattr, we_ref[...], preferred_element_type=jnp.float32)
    xl = xl_ref[...]
    m = xl + xr_ref[...] + eat
    m = jnp.where(m > 0, m, 0.2 * m)
    alpha = jnp.sum(m * att_ref[...], axis=1, keepdims=True)
    ex = jnp.exp(alpha)
    num = jnp.concatenate([nlo_ref[...], nhi_ref[...]], axis=1)
    numf = num + ex * xl
    denf = den_ref[...][:, 0:1] + ex
    o = numf / (denf + 1e-16) + bias_ref[...]
    mu = jnp.mean(o, axis=1, keepdims=True)
    var = jnp.mean((o - mu) ** 2, axis=1, keepdims=True)
    out_ref[...] = (o - mu) * jax.lax.rsqrt(var + 1e-5) * g_ref[...] + be_ref[...]


def _node_pass2(nlo, nhi, den16, cnt16, xl2, xr2, We2, att2, bias2, g2, be2):
    full = lambda *s: pl.BlockSpec(s, lambda i: (0,) * len(s))
    blk64 = pl.BlockSpec((_BLK, 64), lambda i: (i, 0))
    blk32 = pl.BlockSpec((_BLK, 32), lambda i: (i, 0))
    blk16 = pl.BlockSpec((_BLK, 16), lambda i: (i, 0))
    return pl.pallas_call(
        _node_pass2_body,
        grid=(_GRID,),
        in_specs=[
            blk32, blk32, blk16, blk16, blk64, blk64,
            full(3, 64), full(1, 64), full(1, 64), full(1, 64), full(1, 64),
        ],
        out_specs=blk64,
        out_shape=jax.ShapeDtypeStruct((N, 64), jnp.float32),
    )(nlo, nhi, den16, cnt16, xl2, xr2,
      We2, att2.reshape(1, 64), bias2.reshape(1, 64), g2.reshape(1, 64),
      be2.reshape(1, 64))


def kernel(x, edge_attr, edge_index, Wl1, bl1, Wr1, br1, We1, att1, bias1,
           g1, be1, Wl2, bl2, Wr2, br2, We2, att2, bias2, g2, be2):
    src = edge_index[0].astype(jnp.int32)
    dst = edge_index[1].astype(jnp.int32)
    pad = E_PAD - E
    srcp = jnp.concatenate([src, jnp.zeros((pad,), jnp.int32)])
    dstp = jnp.concatenate([dst, jnp.full((pad,), N, jnp.int32)])
    eap = jnp.concatenate(
        [jnp.concatenate([edge_attr,
                          jnp.zeros((E, 1), jnp.float32)], axis=1),
         jnp.zeros((pad, 4), jnp.float32)], axis=0)
    ea4 = eap.reshape(E_PAD // 4, 16)
    zero32 = jnp.zeros((N_PAD, 32), jnp.float32)
    zero16 = jnp.zeros((N_PAD, 16), jnp.float32)

    xl1, xr1 = _node_proj(x, Wl1, bl1, Wr1, br1)

    const1 = jnp.concatenate([We1, att1.reshape(1, 64)], axis=0)
    const2 = jnp.concatenate([We2, att2.reshape(1, 64)], axis=0)

    ep1 = _make_edge_pass(4)
    nlo1, nhi1, exd1 = ep1(srcp, dstp, ea4, xl1, xr1, const1, zero32)
    den1, cnt1 = _make_extra_pass(True)(dstp, exd1, ea4, zero16)

    xl2, xr2 = _node_pass1(nlo1, nhi1, den1, cnt1, xl1, xr1, We1, att1,
                           bias1, g1, be1, Wl2, bl2, Wr2, br2)

    ep2 = _make_edge_pass(1)
    nlo2, nhi2, exd2 = ep2(srcp, dstp, ea4, xl2, xr2, const2, zero32)
    den2, _unused = _make_extra_pass(False)(dstp, exd2, ea4, zero16)

    return _node_pass2(nlo2, nhi2, den2, cnt1, xl2, xr2, We2, att2,
                       bias2, g2, be2)
